# baseline (speedup 1.0000x reference)
import jax
import jax.numpy as jnp
from jax.experimental import pallas as pl
from jax.experimental.pallas import tpu as pltpu

_N = 10000
_BM = 200


def _sage_kernel(adj_ref, x_ref, xi_ref, w1t_ref, w2t_ref, out_ref):
    a = adj_ref[...]
    s = jnp.dot(a, x_ref[...], preferred_element_type=jnp.float32)
    deg = jnp.sum(a, axis=1, keepdims=True)
    neigh = s / (deg + 1.0)
    h = jnp.dot(xi_ref[...], w1t_ref[...], preferred_element_type=jnp.float32)
    h += jnp.dot(neigh, w2t_ref[...], preferred_element_type=jnp.float32)
    out_ref[...] = jnp.maximum(h, 0.0)


@jax.jit
def kernel(x, adj, W):
    nfeat = x.shape[1]
    nembed = W.shape[0]
    w1t = W[:, :nfeat].T
    w2t = W[:, nfeat:].T

    return pl.pallas_call(
        _sage_kernel,
        grid=(_N // _BM,),
        in_specs=[
            pl.BlockSpec((_BM, _N), lambda i: (i, 0)),
            pl.BlockSpec((_N, nfeat), lambda i: (0, 0)),
            pl.BlockSpec((_BM, nfeat), lambda i: (i, 0)),
            pl.BlockSpec((nfeat, nembed), lambda i: (0, 0)),
            pl.BlockSpec((nfeat, nembed), lambda i: (0, 0)),
        ],
        out_specs=pl.BlockSpec((_BM, nembed), lambda i: (i, 0)),
        out_shape=jax.ShapeDtypeStruct((_N, nembed), jnp.float32),
        compiler_params=pltpu.CompilerParams(
            dimension_semantics=("arbitrary",),
        ),
    )(adj, x, x, w1t, w2t)


# speedup vs baseline: 1.0177x; 1.0177x over previous
import jax
import jax.numpy as jnp
from jax.experimental import pallas as pl
from jax.experimental.pallas import tpu as pltpu

_N = 10000
_BM = 400


def _sage_kernel(adj_ref, x_ref, xi_ref, w1t_ref, w2t_ref, out_ref):
    a = adj_ref[...]
    s = jnp.dot(a.astype(jnp.bfloat16), x_ref[...].astype(jnp.bfloat16),
                preferred_element_type=jnp.float32)
    deg = jnp.sum(a, axis=1, keepdims=True)
    neigh = s / (deg + 1.0)
    h = jnp.dot(xi_ref[...], w1t_ref[...], preferred_element_type=jnp.float32)
    h += jnp.dot(neigh, w2t_ref[...], preferred_element_type=jnp.float32)
    out_ref[...] = jnp.maximum(h, 0.0)


@jax.jit
def kernel(x, adj, W):
    nfeat = x.shape[1]
    nembed = W.shape[0]
    w1t = W[:, :nfeat].T
    w2t = W[:, nfeat:].T

    return pl.pallas_call(
        _sage_kernel,
        grid=(_N // _BM,),
        in_specs=[
            pl.BlockSpec((_BM, _N), lambda i: (i, 0)),
            pl.BlockSpec((_N, nfeat), lambda i: (0, 0)),
            pl.BlockSpec((_BM, nfeat), lambda i: (i, 0)),
            pl.BlockSpec((nfeat, nembed), lambda i: (0, 0)),
            pl.BlockSpec((nfeat, nembed), lambda i: (0, 0)),
        ],
        out_specs=pl.BlockSpec((_BM, nembed), lambda i: (i, 0)),
        out_shape=jax.ShapeDtypeStruct((_N, nembed), jnp.float32),
        compiler_params=pltpu.CompilerParams(
            dimension_semantics=("arbitrary",),
        ),
    )(adj, x, x, w1t, w2t)


# speedup vs baseline: 1.0186x; 1.0009x over previous
import jax
import jax.numpy as jnp
from jax.experimental import pallas as pl
from jax.experimental.pallas import tpu as pltpu

_N = 10000
_BM = 400


def _sage_kernel(adj_ref, x_ref, xi_ref, w1t_ref, w2t_ref, out_ref):
    a = adj_ref[...]
    s = jnp.dot(a.astype(jnp.bfloat16), x_ref[...].astype(jnp.bfloat16),
                preferred_element_type=jnp.float32)
    deg = jnp.sum(a, axis=1, keepdims=True)
    neigh = s / (deg + 1.0)
    h = jnp.dot(xi_ref[...], w1t_ref[...], preferred_element_type=jnp.float32)
    h += jnp.dot(neigh, w2t_ref[...], preferred_element_type=jnp.float32)
    out_ref[...] = jnp.maximum(h, 0.0)


@jax.jit
def kernel(x, adj, W):
    nfeat = x.shape[1]
    nembed = W.shape[0]
    w1t = W[:, :nfeat].T
    w2t = W[:, nfeat:].T

    return pl.pallas_call(
        _sage_kernel,
        grid=(_N // _BM,),
        in_specs=[
            pl.BlockSpec((_BM, _N), lambda i: (i, 0)),
            pl.BlockSpec((_N, nfeat), lambda i: (0, 0)),
            pl.BlockSpec((_BM, nfeat), lambda i: (i, 0)),
            pl.BlockSpec((nfeat, nembed), lambda i: (0, 0)),
            pl.BlockSpec((nfeat, nembed), lambda i: (0, 0)),
        ],
        out_specs=pl.BlockSpec((_BM, nembed), lambda i: (i, 0)),
        out_shape=jax.ShapeDtypeStruct((_N, nembed), jnp.float32),
        compiler_params=pltpu.CompilerParams(
            dimension_semantics=("parallel",),
        ),
    )(adj, x, x, w1t, w2t)
